# Initial kernel scaffold; baseline (speedup 1.0000x reference)
import jax
import jax.numpy as jnp
from jax.experimental import pallas as pl


def kernel(hidden_states, Wr, W1, W2, ln_scale, ln_bias):
    raise NotImplementedError("write your pallas kernel here")


# speedup vs baseline: 1.3070x; 1.3070x over previous
import functools

import jax
import jax.numpy as jnp
from jax.experimental import pallas as pl
from jax.experimental.pallas import tpu as pltpu

B, S, D = 1, 2048, 768
E, K, F = 8, 2, 2048
T = B * S
T_TILE = 256
N_T = T // T_TILE


def _moe_dense_kernel(hid_ref, wr_ref, w1_ref, w2_ref, lns_ref, lnb_ref,
                      out_ref, acc_ref):
    e = pl.program_id(0)
    t = pl.program_id(1)

    x = hid_ref[...]
    mu = jnp.mean(x, axis=-1, keepdims=True)
    xc = x - mu
    var = jnp.mean(xc * xc, axis=-1, keepdims=True)
    h = xc * jax.lax.rsqrt(var + 1e-5) * lns_ref[...] + lnb_ref[...]

    logits = jnp.dot(h, wr_ref[...], preferred_element_type=jnp.float32)
    logits = logits - jnp.max(logits, axis=-1, keepdims=True)
    ex = jnp.exp(logits)
    probs = ex / jnp.sum(ex, axis=-1, keepdims=True)

    m1 = jnp.max(probs, axis=-1, keepdims=True)
    probs_wo = jnp.where(probs >= m1, -jnp.inf, probs)
    m2 = jnp.max(probs_wo, axis=-1, keepdims=True)

    esel = jax.lax.broadcasted_iota(jnp.int32, (T_TILE, E), 1) == e
    pe = jnp.sum(jnp.where(esel, probs, 0.0), axis=-1, keepdims=True)
    gate = jnp.where(pe >= m2, pe, 0.0) / (m1 + m2)

    hb = h.astype(jnp.bfloat16)
    mid = jnp.dot(hb, w1_ref[0].astype(jnp.bfloat16),
                  preferred_element_type=jnp.float32)
    mid = mid * jax.nn.sigmoid(mid)
    y = jnp.dot(mid.astype(jnp.bfloat16), w2_ref[0].astype(jnp.bfloat16),
                preferred_element_type=jnp.float32)

    sl = pl.ds(t * T_TILE, T_TILE)

    @pl.when(e == 0)
    def _():
        acc_ref[sl, :] = x + gate * y

    @pl.when(e > 0)
    def _():
        acc_ref[sl, :] += gate * y

    out_ref[...] = acc_ref[sl, :]


@jax.jit
def kernel(hidden_states, Wr, W1, W2, ln_scale, ln_bias):
    b, s, d = hidden_states.shape
    tokens = hidden_states.reshape(-1, d)
    lns = ln_scale.reshape(1, d)
    lnb = ln_bias.reshape(1, d)

    out = pl.pallas_call(
        _moe_dense_kernel,
        grid=(E, N_T),
        in_specs=[
            pl.BlockSpec((T_TILE, D), lambda e, t: (t, 0)),
            pl.BlockSpec((D, E), lambda e, t: (0, 0)),
            pl.BlockSpec((1, D, F), lambda e, t: (e, 0, 0)),
            pl.BlockSpec((1, F, D), lambda e, t: (e, 0, 0)),
            pl.BlockSpec((1, D), lambda e, t: (0, 0)),
            pl.BlockSpec((1, D), lambda e, t: (0, 0)),
        ],
        out_specs=pl.BlockSpec((T_TILE, D), lambda e, t: (t, 0)),
        out_shape=jax.ShapeDtypeStruct((T, D), jnp.float32),
        scratch_shapes=[pltpu.VMEM((T, D), jnp.float32)],
    )(tokens, Wr, W1, W2, lns, lnb)
    return out.reshape(b, s, d)


# speedup vs baseline: 1.7737x; 1.3571x over previous
import functools

import jax
import jax.numpy as jnp
from jax import lax
from jax.experimental import pallas as pl
from jax.experimental.pallas import tpu as pltpu
from jax.experimental.pallas import tpu_sc as plsc

B, S, D = 1, 2048, 768
E, K, F = 8, 2, 2048
T = B * S
A = T * K
TM = 256
MT = A // TM + E
A_PAD = MT * TM
DP = D
N_T = T // TM

_SC_WORKERS = 32
_BPW = A // _SC_WORKERS


def _ln_router(x, wr, lns, lnb):
    mu = jnp.mean(x, axis=-1, keepdims=True)
    xc = x - mu
    var = jnp.mean(xc * xc, axis=-1, keepdims=True)
    h = xc * lax.rsqrt(var + 1e-5) * lns + lnb
    logits = jnp.dot(h, wr, preferred_element_type=jnp.float32)
    logits = logits - jnp.max(logits, axis=-1, keepdims=True)
    ex = jnp.exp(logits)
    probs = ex / jnp.sum(ex, axis=-1, keepdims=True)
    m1 = jnp.max(probs, axis=-1, keepdims=True)
    eio = lax.broadcasted_iota(jnp.int32, probs.shape, 1)
    i1 = jnp.min(jnp.where(probs >= m1, eio, E), axis=-1, keepdims=True)
    mask1 = eio == i1
    probs_wo = jnp.where(mask1, -1.0, probs)
    m2 = jnp.max(probs_wo, axis=-1, keepdims=True)
    i2 = jnp.min(jnp.where(probs_wo >= m2, eio, E), axis=-1, keepdims=True)
    mask2 = eio == i2
    return h, m1, m2, mask1, mask2


def _route_kernel(x_ref, wr_ref, lns_ref, lnb_ref,
                  h_ref, pos_ref, meta_ref):
    x = x_ref[...]
    h, _, _, mask1, mask2 = _ln_router(x, wr_ref[...], lns_ref[...],
                                       lnb_ref[...])
    h_ref[...] = h

    m01 = (mask1 | mask2).astype(jnp.float32)

    CH = 256
    rio = lax.broadcasted_iota(jnp.int32, (CH, CH), 0)
    cio = lax.broadcasted_iota(jnp.int32, (CH, CH), 1)
    ltri = (cio < rio).astype(jnp.bfloat16)
    carry = jnp.zeros((1, E), jnp.float32)
    rank_chunks = []
    for c in range(T // CH):
        mb = m01[c * CH:(c + 1) * CH, :]
        ranks_c = jnp.dot(ltri, mb.astype(jnp.bfloat16),
                          preferred_element_type=jnp.float32) + carry
        rank_chunks.append(ranks_c)
        carry = carry + jnp.sum(mb, axis=0, keepdims=True)
    ranks = jnp.concatenate(rank_chunks, axis=0)
    counts = carry

    tiles = jnp.floor((counts + (TM - 1)) * (1.0 / TM))
    uio_r = lax.broadcasted_iota(jnp.int32, (E, E), 0)
    uio_c = lax.broadcasted_iota(jnp.int32, (E, E), 1)
    utri = (uio_r < uio_c).astype(jnp.bfloat16)
    start_tile = jnp.dot(tiles.astype(jnp.bfloat16), utri,
                         preferred_element_type=jnp.float32)
    n_tiles = jnp.sum(tiles, axis=-1, keepdims=True)
    start_row = start_tile * TM

    sel1 = jnp.sum(jnp.where(mask1, start_row + ranks, 0.0), axis=-1,
                   keepdims=True)
    sel2 = jnp.sum(jnp.where(mask2, start_row + ranks, 0.0), axis=-1,
                   keepdims=True)
    pos_ref[...] = jnp.concatenate([sel1, sel2], axis=1).astype(jnp.int32)

    end_tile = start_tile + tiles
    mio = lax.broadcasted_iota(jnp.int32, (32, E), 0).astype(jnp.float32)
    texp = jnp.sum((mio >= end_tile).astype(jnp.float32), axis=-1,
                   keepdims=True)
    e_last = jnp.sum((end_tile <= n_tiles - 1.0).astype(jnp.float32),
                     axis=-1, keepdims=True)
    texp = jnp.minimum(texp, e_last)
    sio = lax.broadcasted_iota(jnp.int32, (32, 1), 0)
    meta_col = jnp.where(sio == MT, n_tiles, texp).astype(jnp.int32)
    meta_ref[...] = jnp.broadcast_to(meta_col, (32, 128))


def _route(tokens, Wr, lns, lnb):
    return pl.pallas_call(
        _route_kernel,
        out_shape=(
            jax.ShapeDtypeStruct((T, DP), jnp.float32),
            jax.ShapeDtypeStruct((T, K), jnp.int32),
            jax.ShapeDtypeStruct((32, 128), jnp.int32),
        ),
    )(tokens, Wr, lns, lnb)


def _sc_scatter(h_pad, pos_flat):
    mesh = plsc.VectorSubcoreMesh(core_axis_name="c", subcore_axis_name="s")

    @functools.partial(
        pl.kernel, mesh=mesh,
        out_type=jax.ShapeDtypeStruct((A_PAD, DP), jnp.float32),
        scratch_types=[
            pltpu.VMEM((_BPW,), jnp.int32),
            pltpu.VMEM((_BPW, DP), jnp.float32),
            pltpu.SemaphoreType.DMA,
        ],
    )
    def k(h_hbm, idx_hbm, xg_hbm, idx_v, rows_v, sem):
        wid = lax.axis_index("s") * 2 + lax.axis_index("c")
        base = wid * _BPW
        pltpu.sync_copy(idx_hbm.at[pl.ds(base, _BPW)], idx_v)
        pltpu.sync_copy(h_hbm.at[pl.ds(lax.rem(base, T), _BPW)], rows_v)
        pltpu.async_copy(rows_v, xg_hbm.at[idx_v], sem).wait()

    return k(h_pad, pos_flat)


def _sc_gather(y_g, pos_flat):
    mesh = plsc.VectorSubcoreMesh(core_axis_name="c", subcore_axis_name="s")

    @functools.partial(
        pl.kernel, mesh=mesh,
        out_type=jax.ShapeDtypeStruct((A, DP), jnp.float32),
        scratch_types=[
            pltpu.VMEM((_BPW,), jnp.int32),
            pltpu.VMEM((_BPW, DP), jnp.float32),
            pltpu.SemaphoreType.DMA,
        ],
    )
    def k(yg_hbm, idx_hbm, yp_hbm, idx_v, rows_v, sem):
        wid = lax.axis_index("s") * 2 + lax.axis_index("c")
        base = wid * _BPW
        pltpu.sync_copy(idx_hbm.at[pl.ds(base, _BPW)], idx_v)
        pltpu.async_copy(yg_hbm.at[idx_v], rows_v, sem).wait()
        pltpu.sync_copy(rows_v, yp_hbm.at[pl.ds(base, _BPW)])

    return k(y_g, pos_flat)


def _ffn_kernel(s_ref, x_ref, w1_ref, w2_ref, y_ref):
    i = pl.program_id(0)

    @pl.when(i < s_ref[MT])
    def _():
        xb = x_ref[:, :D].astype(jnp.bfloat16)
        w1 = w1_ref[0].astype(jnp.bfloat16)
        mid = jnp.dot(xb, w1, preferred_element_type=jnp.float32)
        mid = mid * jax.nn.sigmoid(mid)
        y = jnp.dot(mid.astype(jnp.bfloat16), w2_ref[0].astype(jnp.bfloat16),
                    preferred_element_type=jnp.float32)
        y_ref[:, :D] = y


def _ffn(meta, x_g, W1, W2):
    grid_spec = pltpu.PrefetchScalarGridSpec(
        num_scalar_prefetch=1,
        grid=(MT,),
        in_specs=[
            pl.BlockSpec((TM, DP),
                         lambda i, s: (jnp.minimum(i, s[MT] - 1), 0)),
            pl.BlockSpec((1, D, F), lambda i, s: (s[i], 0, 0)),
            pl.BlockSpec((1, F, D), lambda i, s: (s[i], 0, 0)),
        ],
        out_specs=pl.BlockSpec((TM, DP),
                               lambda i, s: (jnp.minimum(i, s[MT] - 1), 0)),
    )
    return pl.pallas_call(
        _ffn_kernel,
        grid_spec=grid_spec,
        out_shape=jax.ShapeDtypeStruct((A_PAD, DP), jnp.float32),
    )(meta, x_g, W1, W2)


def _combine_kernel(x_ref, wr_ref, lns_ref, lnb_ref, ya_ref, yb_ref, out_ref):
    x = x_ref[...]
    _, m1, m2, _, _ = _ln_router(x, wr_ref[...], lns_ref[...], lnb_ref[...])
    denom = m1 + m2
    w1 = m1 / denom
    w2 = m2 / denom
    ya = ya_ref[:, :D]
    yb = yb_ref[:, :D]
    out_ref[...] = x + w1 * ya + w2 * yb


def _combine(tokens, Wr, lns, lnb, y_pair):
    return pl.pallas_call(
        _combine_kernel,
        grid=(N_T,),
        in_specs=[
            pl.BlockSpec((TM, D), lambda t: (t, 0)),
            pl.BlockSpec((D, E), lambda t: (0, 0)),
            pl.BlockSpec((1, D), lambda t: (0, 0)),
            pl.BlockSpec((1, D), lambda t: (0, 0)),
            pl.BlockSpec((TM, DP), lambda t: (t, 0)),
            pl.BlockSpec((TM, DP), lambda t: (t + N_T, 0)),
        ],
        out_specs=pl.BlockSpec((TM, D), lambda t: (t, 0)),
        out_shape=jax.ShapeDtypeStruct((T, D), jnp.float32),
    )(tokens, Wr, lns, lnb, y_pair, y_pair)


@jax.jit
def kernel(hidden_states, Wr, W1, W2, ln_scale, ln_bias):
    b, s, d = hidden_states.shape
    tokens = hidden_states.reshape(T, D)
    lns = ln_scale.reshape(1, D)
    lnb = ln_bias.reshape(1, D)

    h_pad, posw, meta2d = _route(tokens, Wr, lns, lnb)
    pos_flat = posw.T.reshape(A)
    meta = meta2d[:, 0]

    x_g = _sc_scatter(h_pad, pos_flat)
    y_g = _ffn(meta, x_g, W1, W2)
    y_pair = _sc_gather(y_g, pos_flat)
    out = _combine(tokens, Wr, lns, lnb, y_pair)
    return out.reshape(b, s, d)


# speedup vs baseline: 1.9620x; 1.1062x over previous
import functools

import jax
import jax.numpy as jnp
from jax import lax
from jax.experimental import pallas as pl
from jax.experimental.pallas import tpu as pltpu
from jax.experimental.pallas import tpu_sc as plsc

B, S, D = 1, 2048, 768
E, K, F = 8, 2, 2048
T = B * S
A = T * K
TM = 256
MT = A // TM + E
A_PAD = MT * TM
DP = D
N_T = T // TM

_SC_WORKERS = 32
_BPW = A // _SC_WORKERS


def _ln_router(x, wr, lns, lnb):
    mu = jnp.mean(x, axis=-1, keepdims=True)
    xc = x - mu
    var = jnp.mean(xc * xc, axis=-1, keepdims=True)
    h = xc * lax.rsqrt(var + 1e-5) * lns + lnb
    logits = jnp.dot(h, wr, preferred_element_type=jnp.float32)
    logits = logits - jnp.max(logits, axis=-1, keepdims=True)
    ex = jnp.exp(logits)
    probs = ex / jnp.sum(ex, axis=-1, keepdims=True)
    m1 = jnp.max(probs, axis=-1, keepdims=True)
    eio = lax.broadcasted_iota(jnp.int32, probs.shape, 1)
    i1 = jnp.min(jnp.where(probs >= m1, eio, E), axis=-1, keepdims=True)
    mask1 = eio == i1
    probs_wo = jnp.where(mask1, -1.0, probs)
    m2 = jnp.max(probs_wo, axis=-1, keepdims=True)
    i2 = jnp.min(jnp.where(probs_wo >= m2, eio, E), axis=-1, keepdims=True)
    mask2 = eio == i2
    return h, m1, m2, mask1, mask2


def _route_kernel(x_ref, wr_ref, lns_ref, lnb_ref,
                  h_ref, pos_ref, meta_ref):
    x = x_ref[...]
    h, _, _, mask1, mask2 = _ln_router(x, wr_ref[...], lns_ref[...],
                                       lnb_ref[...])
    h_ref[...] = h

    m01 = (mask1 | mask2).astype(jnp.float32)

    CH = 256
    rio = lax.broadcasted_iota(jnp.int32, (CH, CH), 0)
    cio = lax.broadcasted_iota(jnp.int32, (CH, CH), 1)
    ltri = (cio < rio).astype(jnp.bfloat16)
    carry = jnp.zeros((1, E), jnp.float32)
    rank_chunks = []
    for c in range(T // CH):
        mb = m01[c * CH:(c + 1) * CH, :]
        ranks_c = jnp.dot(ltri, mb.astype(jnp.bfloat16),
                          preferred_element_type=jnp.float32) + carry
        rank_chunks.append(ranks_c)
        carry = carry + jnp.sum(mb, axis=0, keepdims=True)
    ranks = jnp.concatenate(rank_chunks, axis=0)
    counts = carry

    tiles = jnp.floor((counts + (TM - 1)) * (1.0 / TM))
    uio_r = lax.broadcasted_iota(jnp.int32, (E, E), 0)
    uio_c = lax.broadcasted_iota(jnp.int32, (E, E), 1)
    utri = (uio_r < uio_c).astype(jnp.bfloat16)
    start_tile = jnp.dot(tiles.astype(jnp.bfloat16), utri,
                         preferred_element_type=jnp.float32)
    n_tiles = jnp.sum(tiles, axis=-1, keepdims=True)
    start_row = start_tile * TM

    sel1 = jnp.sum(jnp.where(mask1, start_row + ranks, 0.0), axis=-1,
                   keepdims=True)
    sel2 = jnp.sum(jnp.where(mask2, start_row + ranks, 0.0), axis=-1,
                   keepdims=True)
    pos_ref[...] = jnp.concatenate([sel1, sel2], axis=1).astype(jnp.int32)

    end_tile = start_tile + tiles
    live = (tiles > 0.0).astype(jnp.float32)
    mio = lax.broadcasted_iota(jnp.int32, (32, E), 0).astype(jnp.float32)
    texp = jnp.sum((mio >= end_tile).astype(jnp.float32), axis=-1,
                   keepdims=True)
    e_last = jnp.sum((end_tile <= n_tiles - 1.0).astype(jnp.float32),
                     axis=-1, keepdims=True)
    texp = jnp.minimum(texp, e_last)
    eio = lax.broadcasted_iota(jnp.int32, (32, E), 1).astype(jnp.float32)
    eq = (eio == texp).astype(jnp.float32)
    f_cur = jnp.sum(eq * start_tile, axis=-1, keepdims=True)
    f_next = jnp.sum(eq * end_tile, axis=-1, keepdims=True)
    ordv = jnp.sum((mio >= end_tile).astype(jnp.float32) * live, axis=-1,
                   keepdims=True)
    sio = lax.broadcasted_iota(jnp.int32, (32, 1), 0)
    texp = jnp.where(sio == MT, n_tiles, texp)
    meta_col = jnp.concatenate([texp, f_cur, f_next, ordv],
                               axis=0).astype(jnp.int32)
    meta_ref[...] = jnp.broadcast_to(meta_col, (128, 128))


def _route(tokens, Wr, lns, lnb):
    return pl.pallas_call(
        _route_kernel,
        out_shape=(
            jax.ShapeDtypeStruct((T, DP), jnp.float32),
            jax.ShapeDtypeStruct((T, K), jnp.int32),
            jax.ShapeDtypeStruct((128, 128), jnp.int32),
        ),
    )(tokens, Wr, lns, lnb)


def _sc_scatter(h_pad, pos_flat):
    mesh = plsc.VectorSubcoreMesh(core_axis_name="c", subcore_axis_name="s")

    @functools.partial(
        pl.kernel, mesh=mesh,
        out_type=jax.ShapeDtypeStruct((A_PAD, DP), jnp.float32),
        scratch_types=[
            pltpu.VMEM((_BPW,), jnp.int32),
            pltpu.VMEM((_BPW, DP), jnp.float32),
            pltpu.SemaphoreType.DMA,
        ],
    )
    def k(h_hbm, idx_hbm, xg_hbm, idx_v, rows_v, sem):
        wid = lax.axis_index("s") * 2 + lax.axis_index("c")
        base = wid * _BPW
        pltpu.sync_copy(idx_hbm.at[pl.ds(base, _BPW)], idx_v)
        pltpu.sync_copy(h_hbm.at[pl.ds(lax.rem(base, T), _BPW)], rows_v)
        pltpu.async_copy(rows_v, xg_hbm.at[idx_v], sem).wait()

    return k(h_pad, pos_flat)


def _sc_gather(y_g, pos_flat):
    mesh = plsc.VectorSubcoreMesh(core_axis_name="c", subcore_axis_name="s")

    @functools.partial(
        pl.kernel, mesh=mesh,
        out_type=jax.ShapeDtypeStruct((A, DP), jnp.float32),
        scratch_types=[
            pltpu.VMEM((_BPW,), jnp.int32),
            pltpu.VMEM((_BPW, DP), jnp.float32),
            pltpu.SemaphoreType.DMA,
        ],
    )
    def k(yg_hbm, idx_hbm, yp_hbm, idx_v, rows_v, sem):
        wid = lax.axis_index("s") * 2 + lax.axis_index("c")
        base = wid * _BPW
        pltpu.sync_copy(idx_hbm.at[pl.ds(base, _BPW)], idx_v)
        pltpu.async_copy(yg_hbm.at[idx_v], rows_v, sem).wait()
        pltpu.sync_copy(rows_v, yp_hbm.at[pl.ds(base, _BPW)])

    return k(y_g, pos_flat)


_LOOKAHEAD = 3


def _ffn_kernel(s_ref, x_ref, w1_hbm, w2_hbm, y_ref,
                w1b, w2b, sem1, sem2):
    i = pl.program_id(0)
    n = s_ref[MT]

    @pl.when(i < n)
    def _():
        e_cur = s_ref[i]
        f_cur = s_ref[32 + i]
        f_nxt = s_ref[64 + i]
        ordv = s_ref[96 + i]
        slot = lax.rem(ordv, 2)
        nslot = 1 - slot

        @pl.when(i == 0)
        def _():
            pltpu.make_async_copy(w1_hbm.at[e_cur], w1b.at[slot],
                                  sem1.at[slot]).start()
            pltpu.make_async_copy(w2_hbm.at[e_cur], w2b.at[slot],
                                  sem2.at[slot]).start()

        issue_at = jnp.maximum(f_nxt - _LOOKAHEAD, f_cur)

        @pl.when((i == issue_at) & (f_nxt < n))
        def _():
            e_nxt = s_ref[jnp.minimum(f_nxt, MT - 1)]
            pltpu.make_async_copy(w1_hbm.at[e_nxt], w1b.at[nslot],
                                  sem1.at[nslot]).start()
            pltpu.make_async_copy(w2_hbm.at[e_nxt], w2b.at[nslot],
                                  sem2.at[nslot]).start()

        @pl.when(i == f_cur)
        def _():
            pltpu.make_async_copy(w1_hbm.at[e_cur], w1b.at[slot],
                                  sem1.at[slot]).wait()
            pltpu.make_async_copy(w2_hbm.at[e_cur], w2b.at[slot],
                                  sem2.at[slot]).wait()

        xb = x_ref[:, :D].astype(jnp.bfloat16)
        w1 = w1b[slot].astype(jnp.bfloat16)
        mid = jnp.dot(xb, w1, preferred_element_type=jnp.float32)
        mid = mid * jax.nn.sigmoid(mid)
        y = jnp.dot(mid.astype(jnp.bfloat16), w2b[slot].astype(jnp.bfloat16),
                    preferred_element_type=jnp.float32)
        y_ref[:, :D] = y


def _ffn(meta, x_g, W1, W2):
    grid_spec = pltpu.PrefetchScalarGridSpec(
        num_scalar_prefetch=1,
        grid=(MT,),
        in_specs=[
            pl.BlockSpec((TM, DP),
                         lambda i, s: (jnp.minimum(i, s[MT] - 1), 0)),
            pl.BlockSpec(memory_space=pltpu.MemorySpace.HBM),
            pl.BlockSpec(memory_space=pltpu.MemorySpace.HBM),
        ],
        out_specs=pl.BlockSpec((TM, DP),
                               lambda i, s: (jnp.minimum(i, s[MT] - 1), 0)),
        scratch_shapes=[
            pltpu.VMEM((2, D, F), jnp.float32),
            pltpu.VMEM((2, F, D), jnp.float32),
            pltpu.SemaphoreType.DMA((2,)),
            pltpu.SemaphoreType.DMA((2,)),
        ],
    )
    return pl.pallas_call(
        _ffn_kernel,
        grid_spec=grid_spec,
        out_shape=jax.ShapeDtypeStruct((A_PAD, DP), jnp.float32),
    )(meta, x_g, W1, W2)


def _combine_kernel(x_ref, wr_ref, lns_ref, lnb_ref, ya_ref, yb_ref, out_ref):
    x = x_ref[...]
    _, m1, m2, _, _ = _ln_router(x, wr_ref[...], lns_ref[...], lnb_ref[...])
    denom = m1 + m2
    w1 = m1 / denom
    w2 = m2 / denom
    ya = ya_ref[:, :D]
    yb = yb_ref[:, :D]
    out_ref[...] = x + w1 * ya + w2 * yb


def _combine(tokens, Wr, lns, lnb, y_pair):
    return pl.pallas_call(
        _combine_kernel,
        grid=(N_T,),
        in_specs=[
            pl.BlockSpec((TM, D), lambda t: (t, 0)),
            pl.BlockSpec((D, E), lambda t: (0, 0)),
            pl.BlockSpec((1, D), lambda t: (0, 0)),
            pl.BlockSpec((1, D), lambda t: (0, 0)),
            pl.BlockSpec((TM, DP), lambda t: (t, 0)),
            pl.BlockSpec((TM, DP), lambda t: (t + N_T, 0)),
        ],
        out_specs=pl.BlockSpec((TM, D), lambda t: (t, 0)),
        out_shape=jax.ShapeDtypeStruct((T, D), jnp.float32),
    )(tokens, Wr, lns, lnb, y_pair, y_pair)


@jax.jit
def kernel(hidden_states, Wr, W1, W2, ln_scale, ln_bias):
    b, s, d = hidden_states.shape
    tokens = hidden_states.reshape(T, D)
    lns = ln_scale.reshape(1, D)
    lnb = ln_bias.reshape(1, D)

    h_pad, posw, meta2d = _route(tokens, Wr, lns, lnb)
    pos_flat = posw.T.reshape(A)
    meta = meta2d[:, 0]

    x_g = _sc_scatter(h_pad, pos_flat)
    y_g = _ffn(meta, x_g, W1, W2)
    y_pair = _sc_gather(y_g, pos_flat)
    out = _combine(tokens, Wr, lns, lnb, y_pair)
    return out.reshape(b, s, d)


# speedup vs baseline: 2.2375x; 1.1405x over previous
import functools

import jax
import jax.numpy as jnp
from jax import lax
from jax.experimental import pallas as pl
from jax.experimental.pallas import tpu as pltpu
from jax.experimental.pallas import tpu_sc as plsc

B, S, D = 1, 2048, 768
E, K, F = 8, 2, 2048
T = B * S
A = T * K
TM = 256
MT = A // TM + E
A_PAD = MT * TM
DH = D // 2
N_T = T // TM

_SC_WORKERS = 32
_BPW = A // _SC_WORKERS


def _pack2(y):
    yl = lax.bitcast_convert_type(y[..., :DH], jnp.int32)
    yh = lax.bitcast_convert_type(y[..., DH:], jnp.int32)
    lo = jnp.right_shift(yl + 0x8000, 16) & jnp.int32(0xFFFF)
    hi = (yh + 0x8000) & jnp.int32(-65536)
    return lo | hi


def _unpack2(u):
    lo = lax.bitcast_convert_type(jnp.left_shift(u, 16), jnp.float32)
    hi = lax.bitcast_convert_type(u & jnp.int32(-65536), jnp.float32)
    return jnp.concatenate([lo, hi], axis=-1)


def _ln_router(x, wr, lns, lnb):
    mu = jnp.mean(x, axis=-1, keepdims=True)
    xc = x - mu
    var = jnp.mean(xc * xc, axis=-1, keepdims=True)
    h = xc * lax.rsqrt(var + 1e-5) * lns + lnb
    logits = jnp.dot(h, wr, preferred_element_type=jnp.float32)
    logits = logits - jnp.max(logits, axis=-1, keepdims=True)
    ex = jnp.exp(logits)
    probs = ex / jnp.sum(ex, axis=-1, keepdims=True)
    m1 = jnp.max(probs, axis=-1, keepdims=True)
    eio = lax.broadcasted_iota(jnp.int32, probs.shape, 1)
    i1 = jnp.min(jnp.where(probs >= m1, eio, E), axis=-1, keepdims=True)
    mask1 = eio == i1
    probs_wo = jnp.where(mask1, -1.0, probs)
    m2 = jnp.max(probs_wo, axis=-1, keepdims=True)
    i2 = jnp.min(jnp.where(probs_wo >= m2, eio, E), axis=-1, keepdims=True)
    mask2 = eio == i2
    return h, m1, m2, mask1, mask2


def _route_kernel(x_ref, wr_ref, lns_ref, lnb_ref,
                  h_ref, pos_ref, w_ref, meta_ref):
    x = x_ref[...]
    h, m1, m2, mask1, mask2 = _ln_router(x, wr_ref[...], lns_ref[...],
                                         lnb_ref[...])
    h_ref[...] = _pack2(h)
    denom = m1 + m2
    w_ref[...] = jnp.concatenate([m1 / denom, m2 / denom], axis=1)

    m01 = (mask1 | mask2).astype(jnp.float32)

    CH = 256
    rio = lax.broadcasted_iota(jnp.int32, (CH, CH), 0)
    cio = lax.broadcasted_iota(jnp.int32, (CH, CH), 1)
    ltri = (cio < rio).astype(jnp.bfloat16)
    carry = jnp.zeros((1, E), jnp.float32)
    rank_chunks = []
    for c in range(T // CH):
        mb = m01[c * CH:(c + 1) * CH, :]
        ranks_c = jnp.dot(ltri, mb.astype(jnp.bfloat16),
                          preferred_element_type=jnp.float32) + carry
        rank_chunks.append(ranks_c)
        carry = carry + jnp.sum(mb, axis=0, keepdims=True)
    ranks = jnp.concatenate(rank_chunks, axis=0)
    counts = carry

    tiles = jnp.floor((counts + (TM - 1)) * (1.0 / TM))
    uio_r = lax.broadcasted_iota(jnp.int32, (E, E), 0)
    uio_c = lax.broadcasted_iota(jnp.int32, (E, E), 1)
    utri = (uio_r < uio_c).astype(jnp.bfloat16)
    start_tile = jnp.dot(tiles.astype(jnp.bfloat16), utri,
                         preferred_element_type=jnp.float32)
    n_tiles = jnp.sum(tiles, axis=-1, keepdims=True)
    start_row = start_tile * TM

    sel1 = jnp.sum(jnp.where(mask1, start_row + ranks, 0.0), axis=-1,
                   keepdims=True)
    sel2 = jnp.sum(jnp.where(mask2, start_row + ranks, 0.0), axis=-1,
                   keepdims=True)
    pos_ref[...] = jnp.concatenate([sel1, sel2], axis=1).astype(jnp.int32)

    end_tile = start_tile + tiles
    live = (tiles > 0.0).astype(jnp.float32)
    mio = lax.broadcasted_iota(jnp.int32, (32, E), 0).astype(jnp.float32)
    texp = jnp.sum((mio >= end_tile).astype(jnp.float32), axis=-1,
                   keepdims=True)
    e_last = jnp.sum((end_tile <= n_tiles - 1.0).astype(jnp.float32),
                     axis=-1, keepdims=True)
    texp = jnp.minimum(texp, e_last)
    eio = lax.broadcasted_iota(jnp.int32, (32, E), 1).astype(jnp.float32)
    eq = (eio == texp).astype(jnp.float32)
    f_cur = jnp.sum(eq * start_tile, axis=-1, keepdims=True)
    f_next = jnp.sum(eq * end_tile, axis=-1, keepdims=True)
    ordv = jnp.sum((mio >= end_tile).astype(jnp.float32) * live, axis=-1,
                   keepdims=True)
    sio = lax.broadcasted_iota(jnp.int32, (32, 1), 0)
    texp = jnp.where(sio == MT, n_tiles, texp)
    meta_col = jnp.concatenate([texp, f_cur, f_next, ordv],
                               axis=0).astype(jnp.int32)
    meta_ref[...] = jnp.broadcast_to(meta_col, (128, 128))


def _route(tokens, Wr, lns, lnb):
    return pl.pallas_call(
        _route_kernel,
        out_shape=(
            jax.ShapeDtypeStruct((T, DH), jnp.int32),
            jax.ShapeDtypeStruct((T, K), jnp.int32),
            jax.ShapeDtypeStruct((T, K), jnp.float32),
            jax.ShapeDtypeStruct((128, 128), jnp.int32),
        ),
    )(tokens, Wr, lns, lnb)


def _sc_scatter(h_packed, pos_flat):
    mesh = plsc.VectorSubcoreMesh(core_axis_name="c", subcore_axis_name="s")

    @functools.partial(
        pl.kernel, mesh=mesh,
        out_type=jax.ShapeDtypeStruct((A_PAD, DH), jnp.int32),
        scratch_types=[
            pltpu.VMEM((_BPW,), jnp.int32),
            pltpu.VMEM((_BPW, DH), jnp.int32),
            pltpu.SemaphoreType.DMA,
        ],
    )
    def k(h_hbm, idx_hbm, xg_hbm, idx_v, rows_v, sem):
        wid = lax.axis_index("s") * 2 + lax.axis_index("c")
        base = wid * _BPW
        pltpu.sync_copy(idx_hbm.at[pl.ds(base, _BPW)], idx_v)
        pltpu.sync_copy(h_hbm.at[pl.ds(lax.rem(base, T), _BPW)], rows_v)
        pltpu.async_copy(rows_v, xg_hbm.at[idx_v], sem).wait()

    return k(h_packed, pos_flat)


def _sc_gather(y_g, pos_flat):
    mesh = plsc.VectorSubcoreMesh(core_axis_name="c", subcore_axis_name="s")

    @functools.partial(
        pl.kernel, mesh=mesh,
        out_type=jax.ShapeDtypeStruct((A, DH), jnp.int32),
        scratch_types=[
            pltpu.VMEM((_BPW,), jnp.int32),
            pltpu.VMEM((_BPW, DH), jnp.int32),
            pltpu.SemaphoreType.DMA,
        ],
    )
    def k(yg_hbm, idx_hbm, yp_hbm, idx_v, rows_v, sem):
        wid = lax.axis_index("s") * 2 + lax.axis_index("c")
        base = wid * _BPW
        pltpu.sync_copy(idx_hbm.at[pl.ds(base, _BPW)], idx_v)
        pltpu.async_copy(yg_hbm.at[idx_v], rows_v, sem).wait()
        pltpu.sync_copy(rows_v, yp_hbm.at[pl.ds(base, _BPW)])

    return k(y_g, pos_flat)


_LOOKAHEAD = 3


def _ffn_kernel(s_ref, x_ref, w1_hbm, w2_hbm, y_ref,
                w1b, w2b, sem1, sem2):
    i = pl.program_id(0)
    n = s_ref[MT]

    @pl.when(i < n)
    def _():
        e_cur = s_ref[i]
        f_cur = s_ref[32 + i]
        f_nxt = s_ref[64 + i]
        ordv = s_ref[96 + i]
        slot = lax.rem(ordv, 2)
        nslot = 1 - slot

        @pl.when(i == 0)
        def _():
            pltpu.make_async_copy(w1_hbm.at[e_cur], w1b.at[slot],
                                  sem1.at[slot]).start()
            pltpu.make_async_copy(w2_hbm.at[e_cur], w2b.at[slot],
                                  sem2.at[slot]).start()

        issue_at = jnp.maximum(f_nxt - _LOOKAHEAD, f_cur)

        @pl.when((i == issue_at) & (f_nxt < n))
        def _():
            e_nxt = s_ref[jnp.minimum(f_nxt, MT - 1)]
            pltpu.make_async_copy(w1_hbm.at[e_nxt], w1b.at[nslot],
                                  sem1.at[nslot]).start()
            pltpu.make_async_copy(w2_hbm.at[e_nxt], w2b.at[nslot],
                                  sem2.at[nslot]).start()

        @pl.when(i == f_cur)
        def _():
            pltpu.make_async_copy(w1_hbm.at[e_cur], w1b.at[slot],
                                  sem1.at[slot]).wait()
            pltpu.make_async_copy(w2_hbm.at[e_cur], w2b.at[slot],
                                  sem2.at[slot]).wait()

        xb = _unpack2(x_ref[...]).astype(jnp.bfloat16)
        w1 = w1b[slot].astype(jnp.bfloat16)
        mid = jnp.dot(xb, w1, preferred_element_type=jnp.float32)
        mid = mid * jax.nn.sigmoid(mid)
        y = jnp.dot(mid.astype(jnp.bfloat16), w2b[slot].astype(jnp.bfloat16),
                    preferred_element_type=jnp.float32)
        y_ref[...] = _pack2(y)


def _ffn(meta, x_g, W1, W2):
    grid_spec = pltpu.PrefetchScalarGridSpec(
        num_scalar_prefetch=1,
        grid=(MT,),
        in_specs=[
            pl.BlockSpec((TM, DH),
                         lambda i, s: (jnp.minimum(i, s[MT] - 1), 0)),
            pl.BlockSpec(memory_space=pltpu.MemorySpace.HBM),
            pl.BlockSpec(memory_space=pltpu.MemorySpace.HBM),
        ],
        out_specs=pl.BlockSpec((TM, DH),
                               lambda i, s: (jnp.minimum(i, s[MT] - 1), 0)),
        scratch_shapes=[
            pltpu.VMEM((2, D, F), jnp.float32),
            pltpu.VMEM((2, F, D), jnp.float32),
            pltpu.SemaphoreType.DMA((2,)),
            pltpu.SemaphoreType.DMA((2,)),
        ],
    )
    return pl.pallas_call(
        _ffn_kernel,
        grid_spec=grid_spec,
        out_shape=jax.ShapeDtypeStruct((A_PAD, DH), jnp.int32),
    )(meta, x_g, W1, W2)


def _combine_kernel(x_ref, w_ref, ya_ref, yb_ref, out_ref):
    w = w_ref[...]
    ya = _unpack2(ya_ref[...])
    yb = _unpack2(yb_ref[...])
    out_ref[...] = x_ref[...] + w[:, 0:1] * ya + w[:, 1:2] * yb


def _combine(tokens, wpair, y_pair):
    return pl.pallas_call(
        _combine_kernel,
        grid=(N_T,),
        in_specs=[
            pl.BlockSpec((TM, D), lambda t: (t, 0)),
            pl.BlockSpec((TM, K), lambda t: (t, 0)),
            pl.BlockSpec((TM, DH), lambda t: (t, 0)),
            pl.BlockSpec((TM, DH), lambda t: (t + N_T, 0)),
        ],
        out_specs=pl.BlockSpec((TM, D), lambda t: (t, 0)),
        out_shape=jax.ShapeDtypeStruct((T, D), jnp.float32),
    )(tokens, wpair, y_pair, y_pair)


@jax.jit
def kernel(hidden_states, Wr, W1, W2, ln_scale, ln_bias):
    b, s, d = hidden_states.shape
    tokens = hidden_states.reshape(T, D)
    lns = ln_scale.reshape(1, D)
    lnb = ln_bias.reshape(1, D)

    h_packed, posw, wpair, meta2d = _route(tokens, Wr, lns, lnb)
    pos_flat = posw.T.reshape(A)
    meta = meta2d[:, 0]

    x_g = _sc_scatter(h_packed, pos_flat)
    y_g = _ffn(meta, x_g, W1, W2)
    y_pair = _sc_gather(y_g, pos_flat)
    out = _combine(tokens, wpair, y_pair)
    return out.reshape(b, s, d)
